# baseline (speedup 1.0000x reference)
import functools

import jax
import jax.numpy as jnp
from jax import lax
from jax.experimental import pallas as pl
from jax.experimental.pallas import tpu as pltpu
from jax.experimental.pallas import tpu_sc as plsc

N_CODES = 8192
D = 256
N_TOK = 8192
BT = 256
N_BLK = N_TOK // BT


_CHUNK_BOUNDS = (0, 4096, 8192)


def _dist_argmin_body(x_ref, wT_ref, xsq_ref, wsq_ref, codes_ref, counts_ref):
    i = pl.program_id(0)
    dot = jnp.dot(x_ref[...], wT_ref[...], preferred_element_type=jnp.float32)
    d = xsq_ref[...] - 2.0 * dot
    d = d + wsq_ref[...]
    iota = lax.broadcasted_iota(jnp.int32, d.shape, 1)
    inf = jnp.float32(jnp.inf)
    acc_v = jnp.full((d.shape[0], 1), inf, jnp.float32)
    acc_i = jnp.zeros((d.shape[0], 1), jnp.int32)
    for a, b in zip(_CHUNK_BOUNDS[:-1], _CHUNK_BOUNDS[1:]):
        in_chunk = (iota >= a) & (iota < b)
        dm = jnp.where(in_chunk, d, inf)
        m_c = jnp.min(dm, axis=1, keepdims=True)
        i_c = jnp.min(jnp.where(dm == m_c, iota, N_CODES), axis=1, keepdims=True)
        win = (m_c < acc_v) | ((m_c == acc_v) & (i_c < acc_i))
        m_bf = m_c.astype(jnp.bfloat16).astype(jnp.float32)
        acc_v = jnp.where(win, m_bf, acc_v)
        acc_i = jnp.where(win, i_c, acc_i)
    idx = acc_i
    codes_ref[...] = idx
    onehot = (iota == idx).astype(jnp.float32)
    blk_counts = jnp.sum(onehot, axis=0, keepdims=True)

    @pl.when(i == 0)
    def _():
        counts_ref[...] = blk_counts

    @pl.when(i > 0)
    def _():
        counts_ref[...] += blk_counts


_dist_argmin = pl.pallas_call(
    _dist_argmin_body,
    grid=(N_BLK,),
    in_specs=[
        pl.BlockSpec((BT, D), lambda i: (i, 0)),
        pl.BlockSpec((D, N_CODES), lambda i: (0, 0)),
        pl.BlockSpec((BT, 1), lambda i: (i, 0)),
        pl.BlockSpec((1, N_CODES), lambda i: (0, 0)),
    ],
    out_specs=[
        pl.BlockSpec((BT, 1), lambda i: (i, 0)),
        pl.BlockSpec((1, N_CODES), lambda i: (0, 0)),
    ],
    out_shape=[
        jax.ShapeDtypeStruct((N_TOK, 1), jnp.int32),
        jax.ShapeDtypeStruct((1, N_CODES), jnp.float32),
    ],
)


def _make_sc_gather():
    info = plsc.get_sparse_core_info()
    nw = info.num_cores * info.num_subcores
    rows_per_w = N_TOK // nw
    chunk = 128
    n_chunks = rows_per_w // chunk
    mesh = plsc.VectorSubcoreMesh(core_axis_name="c", subcore_axis_name="s")

    @functools.partial(
        pl.kernel,
        out_type=jax.ShapeDtypeStruct((N_TOK, D), jnp.float32),
        mesh=mesh,
        scratch_types=[
            pltpu.VMEM((n_chunks, chunk), jnp.int32),
            pltpu.VMEM((rows_per_w, D), jnp.float32),
            pltpu.SemaphoreType.DMA,
        ],
    )
    def gather_kernel(weight_hbm, codes_hbm, out_hbm, idx_v, rows_v, sem):
        wid = lax.axis_index("s") * info.num_cores + lax.axis_index("c")
        base = wid * rows_per_w
        pltpu.sync_copy(
            codes_hbm.at[pl.ds(wid * n_chunks, n_chunks)], idx_v)
        copies = [
            pltpu.async_copy(
                weight_hbm.at[idx_v.at[j]],
                rows_v.at[pl.ds(j * chunk, chunk)],
                sem,
            )
            for j in range(n_chunks)
        ]
        for c in copies:
            c.wait()
        pltpu.sync_copy(rows_v, out_hbm.at[pl.ds(base, rows_per_w)])

    def run(weight, codes_flat):
        codes2d = codes_flat.reshape(N_TOK // chunk, chunk)
        return gather_kernel(weight, codes2d)

    return run


_sc_gather_cache = []


def _sc_gather(weight, codes_flat):
    if not _sc_gather_cache:
        _sc_gather_cache.append(_make_sc_gather())
    return _sc_gather_cache[0](weight, codes_flat)


def _loss_perp_body(x_ref, q_ref, cnt_ref, loss_ref, perp_ref):
    diff = q_ref[...] - x_ref[...]
    s = jnp.sum(diff * diff)
    loss_ref[...] = (1.25 * (s / float(N_TOK * D))).reshape(1, 1)
    p = cnt_ref[...] * (1.0 / N_TOK)
    ent = jnp.sum(p * jnp.log(p + 1e-10))
    perp_ref[...] = jnp.exp(-ent).reshape(1, 1)


_loss_perp = pl.pallas_call(
    _loss_perp_body,
    in_specs=[
        pl.BlockSpec((N_TOK, D), lambda: (0, 0)),
        pl.BlockSpec((N_TOK, D), lambda: (0, 0)),
        pl.BlockSpec((1, N_CODES), lambda: (0, 0)),
    ],
    out_specs=[
        pl.BlockSpec((1, 1), lambda: (0, 0)),
        pl.BlockSpec((1, 1), lambda: (0, 0)),
    ],
    out_shape=[
        jax.ShapeDtypeStruct((1, 1), jnp.float32),
        jax.ShapeDtypeStruct((1, 1), jnp.float32),
    ],
)


def kernel(inputs, weight):
    flat = inputs.reshape(-1, D)
    xsq = jnp.sum(flat**2, axis=1, keepdims=True)
    wsq = jnp.sum(weight**2, axis=1).reshape(1, N_CODES)
    wT = weight.T
    codes, counts = _dist_argmin(flat, wT, xsq, wsq)
    quant_flat = _sc_gather(weight, codes.reshape(-1))
    loss, perp = _loss_perp(flat, quant_flat, counts)
    quantized = quant_flat.reshape(inputs.shape)
    return (quantized, loss[0, 0], perp[0, 0], codes)


# speedup vs baseline: 1.1418x; 1.1418x over previous
import functools

import jax
import jax.numpy as jnp
from jax import lax
from jax.experimental import pallas as pl
from jax.experimental.pallas import tpu as pltpu
from jax.experimental.pallas import tpu_sc as plsc

N_CODES = 8192
D = 256
N_TOK = 8192
BT = 256
N_BLK = N_TOK // BT


_CHUNK_BOUNDS = (0, 4096, 8192)


def _dist_argmin_body(x_ref, wT_ref, xsq_ref, wsq_ref, codes_ref, counts_ref):
    i = pl.program_id(0)
    dot = jnp.dot(x_ref[...], wT_ref[...], preferred_element_type=jnp.float32)
    d = xsq_ref[...] - 2.0 * dot
    d = d + wsq_ref[...]
    iota = lax.broadcasted_iota(jnp.int32, d.shape, 1)
    acc_v = jnp.full((d.shape[0], 1), jnp.float32(jnp.inf), jnp.float32)
    acc_i = jnp.zeros((d.shape[0], 1), jnp.int32)
    for a, b in zip(_CHUNK_BOUNDS[:-1], _CHUNK_BOUNDS[1:]):
        dm = d[:, a:b]
        io = iota[:, a:b]
        m_c = jnp.min(dm, axis=1, keepdims=True)
        i_c = jnp.min(jnp.where(dm == m_c, io, N_CODES), axis=1, keepdims=True)
        win = (m_c < acc_v) | ((m_c == acc_v) & (i_c < acc_i))
        m_bf = m_c.astype(jnp.bfloat16).astype(jnp.float32)
        acc_v = jnp.where(win, m_bf, acc_v)
        acc_i = jnp.where(win, i_c, acc_i)
    idx = acc_i
    codes_ref[...] = idx
    onehot = (iota == idx).astype(jnp.float32)
    blk_counts = jnp.sum(onehot, axis=0, keepdims=True)

    @pl.when(i == 0)
    def _():
        counts_ref[...] = blk_counts

    @pl.when(i > 0)
    def _():
        counts_ref[...] += blk_counts


_dist_argmin = pl.pallas_call(
    _dist_argmin_body,
    grid=(N_BLK,),
    in_specs=[
        pl.BlockSpec((BT, D), lambda i: (i, 0)),
        pl.BlockSpec((D, N_CODES), lambda i: (0, 0)),
        pl.BlockSpec((BT, 1), lambda i: (i, 0)),
        pl.BlockSpec((1, N_CODES), lambda i: (0, 0)),
    ],
    out_specs=[
        pl.BlockSpec((BT, 1), lambda i: (i, 0)),
        pl.BlockSpec((1, N_CODES), lambda i: (0, 0)),
    ],
    out_shape=[
        jax.ShapeDtypeStruct((N_TOK, 1), jnp.int32),
        jax.ShapeDtypeStruct((1, N_CODES), jnp.float32),
    ],
)


def _make_sc_gather():
    info = plsc.get_sparse_core_info()
    nw = info.num_cores * info.num_subcores
    rows_per_w = N_TOK // nw
    chunk = 128
    n_chunks = rows_per_w // chunk
    mesh = plsc.VectorSubcoreMesh(core_axis_name="c", subcore_axis_name="s")

    @functools.partial(
        pl.kernel,
        out_type=jax.ShapeDtypeStruct((N_TOK, D), jnp.float32),
        mesh=mesh,
        scratch_types=[
            pltpu.VMEM((n_chunks, chunk), jnp.int32),
            pltpu.VMEM((rows_per_w, D), jnp.float32),
            pltpu.SemaphoreType.DMA,
        ],
    )
    def gather_kernel(weight_hbm, codes_hbm, out_hbm, idx_v, rows_v, sem):
        wid = lax.axis_index("s") * info.num_cores + lax.axis_index("c")
        base = wid * rows_per_w
        pltpu.sync_copy(
            codes_hbm.at[pl.ds(wid * n_chunks, n_chunks)], idx_v)
        copies = [
            pltpu.async_copy(
                weight_hbm.at[idx_v.at[j]],
                rows_v.at[pl.ds(j * chunk, chunk)],
                sem,
            )
            for j in range(n_chunks)
        ]
        for c in copies:
            c.wait()
        pltpu.sync_copy(rows_v, out_hbm.at[pl.ds(base, rows_per_w)])

    def run(weight, codes_flat):
        codes2d = codes_flat.reshape(N_TOK // chunk, chunk)
        return gather_kernel(weight, codes2d)

    return run


_sc_gather_cache = []


def _sc_gather(weight, codes_flat):
    if not _sc_gather_cache:
        _sc_gather_cache.append(_make_sc_gather())
    return _sc_gather_cache[0](weight, codes_flat)


def _loss_perp_body(x_ref, q_ref, cnt_ref, loss_ref, perp_ref):
    diff = q_ref[...] - x_ref[...]
    s = jnp.sum(diff * diff)
    loss_ref[...] = (1.25 * (s / float(N_TOK * D))).reshape(1, 1)
    p = cnt_ref[...] * (1.0 / N_TOK)
    ent = jnp.sum(p * jnp.log(p + 1e-10))
    perp_ref[...] = jnp.exp(-ent).reshape(1, 1)


_loss_perp = pl.pallas_call(
    _loss_perp_body,
    in_specs=[
        pl.BlockSpec((N_TOK, D), lambda: (0, 0)),
        pl.BlockSpec((N_TOK, D), lambda: (0, 0)),
        pl.BlockSpec((1, N_CODES), lambda: (0, 0)),
    ],
    out_specs=[
        pl.BlockSpec((1, 1), lambda: (0, 0)),
        pl.BlockSpec((1, 1), lambda: (0, 0)),
    ],
    out_shape=[
        jax.ShapeDtypeStruct((1, 1), jnp.float32),
        jax.ShapeDtypeStruct((1, 1), jnp.float32),
    ],
)


def kernel(inputs, weight):
    flat = inputs.reshape(-1, D)
    xsq = jnp.sum(flat**2, axis=1, keepdims=True)
    wsq = jnp.sum(weight**2, axis=1).reshape(1, N_CODES)
    wT = weight.T
    codes, counts = _dist_argmin(flat, wT, xsq, wsq)
    quant_flat = _sc_gather(weight, codes.reshape(-1))
    loss, perp = _loss_perp(flat, quant_flat, counts)
    quantized = quant_flat.reshape(inputs.shape)
    return (quantized, loss[0, 0], perp[0, 0], codes)


# speedup vs baseline: 1.1840x; 1.0369x over previous
import functools

import jax
import jax.numpy as jnp
from jax import lax
from jax.experimental import pallas as pl
from jax.experimental.pallas import tpu as pltpu
from jax.experimental.pallas import tpu_sc as plsc

N_CODES = 8192
D = 256
N_TOK = 8192
BT = 256
N_BLK = N_TOK // BT


_CHUNK_BOUNDS = (0, 4096, 8192)


def _dist_argmin_body(x_ref, wT_ref, xsq_ref, wsq_ref, codes_ref, counts_ref,
                      mdist_ref):
    i = pl.program_id(0)
    dot = jnp.dot(x_ref[...], wT_ref[...], preferred_element_type=jnp.float32)
    d = xsq_ref[...] - 2.0 * dot
    d = d + wsq_ref[...]
    iota = lax.broadcasted_iota(jnp.int32, d.shape, 1)
    acc_v = jnp.full((d.shape[0], 1), jnp.float32(jnp.inf), jnp.float32)
    acc_i = jnp.zeros((d.shape[0], 1), jnp.int32)
    acc_raw = jnp.zeros((d.shape[0], 1), jnp.float32)
    for a, b in zip(_CHUNK_BOUNDS[:-1], _CHUNK_BOUNDS[1:]):
        dm = d[:, a:b]
        io = iota[:, a:b]
        m_c = jnp.min(dm, axis=1, keepdims=True)
        i_c = jnp.min(jnp.where(dm == m_c, io, N_CODES), axis=1, keepdims=True)
        win = (m_c < acc_v) | ((m_c == acc_v) & (i_c < acc_i))
        m_bf = m_c.astype(jnp.bfloat16).astype(jnp.float32)
        acc_v = jnp.where(win, m_bf, acc_v)
        acc_raw = jnp.where(win, m_c, acc_raw)
        acc_i = jnp.where(win, i_c, acc_i)
    idx = acc_i
    codes_ref[...] = idx
    mdist_ref[...] = acc_raw
    onehot = (iota == idx).astype(jnp.float32)
    blk_counts = jnp.sum(onehot, axis=0, keepdims=True)

    @pl.when(i == 0)
    def _():
        counts_ref[...] = blk_counts

    @pl.when(i > 0)
    def _():
        counts_ref[...] += blk_counts


_dist_argmin = pl.pallas_call(
    _dist_argmin_body,
    grid=(N_BLK,),
    in_specs=[
        pl.BlockSpec((BT, D), lambda i: (i, 0)),
        pl.BlockSpec((D, N_CODES), lambda i: (0, 0)),
        pl.BlockSpec((BT, 1), lambda i: (i, 0)),
        pl.BlockSpec((1, N_CODES), lambda i: (0, 0)),
    ],
    out_specs=[
        pl.BlockSpec((BT, 1), lambda i: (i, 0)),
        pl.BlockSpec((1, N_CODES), lambda i: (0, 0)),
        pl.BlockSpec((BT, 1), lambda i: (i, 0)),
    ],
    out_shape=[
        jax.ShapeDtypeStruct((N_TOK, 1), jnp.int32),
        jax.ShapeDtypeStruct((1, N_CODES), jnp.float32),
        jax.ShapeDtypeStruct((N_TOK, 1), jnp.float32),
    ],
)


def _make_sc_gather():
    info = plsc.get_sparse_core_info()
    nw = info.num_cores * info.num_subcores
    rows_per_w = N_TOK // nw
    chunk = 128
    n_chunks = rows_per_w // chunk
    mesh = plsc.VectorSubcoreMesh(core_axis_name="c", subcore_axis_name="s")

    @functools.partial(
        pl.kernel,
        out_type=jax.ShapeDtypeStruct((N_TOK, D), jnp.float32),
        mesh=mesh,
        scratch_types=[
            pltpu.VMEM((n_chunks, chunk), jnp.int32),
            pltpu.VMEM((rows_per_w, D), jnp.float32),
            pltpu.SemaphoreType.DMA,
        ],
    )
    def gather_kernel(weight_hbm, codes_hbm, out_hbm, idx_v, rows_v, sem):
        wid = lax.axis_index("s") * info.num_cores + lax.axis_index("c")
        base = wid * rows_per_w
        pltpu.sync_copy(
            codes_hbm.at[pl.ds(wid * n_chunks, n_chunks)], idx_v)
        copies = [
            pltpu.async_copy(
                weight_hbm.at[idx_v.at[j]],
                rows_v.at[pl.ds(j * chunk, chunk)],
                sem,
            )
            for j in range(n_chunks)
        ]
        for c in copies:
            c.wait()
        pltpu.sync_copy(rows_v, out_hbm.at[pl.ds(base, rows_per_w)])

    def run(weight, codes_flat):
        codes2d = codes_flat.reshape(N_TOK // chunk, chunk)
        return gather_kernel(weight, codes2d)

    return run


_sc_gather_cache = []


def _sc_gather(weight, codes_flat):
    if not _sc_gather_cache:
        _sc_gather_cache.append(_make_sc_gather())
    return _sc_gather_cache[0](weight, codes_flat)


def _loss_perp_body(mdist_ref, cnt_ref, loss_ref, perp_ref):
    s = jnp.sum(mdist_ref[...])
    loss_ref[...] = (1.25 * (s / float(N_TOK * D))).reshape(1, 1)
    p = cnt_ref[...] * (1.0 / N_TOK)
    ent = jnp.sum(p * jnp.log(p + 1e-10))
    perp_ref[...] = jnp.exp(-ent).reshape(1, 1)


_loss_perp = pl.pallas_call(
    _loss_perp_body,
    in_specs=[
        pl.BlockSpec((N_TOK, 1), lambda: (0, 0)),
        pl.BlockSpec((1, N_CODES), lambda: (0, 0)),
    ],
    out_specs=[
        pl.BlockSpec((1, 1), lambda: (0, 0)),
        pl.BlockSpec((1, 1), lambda: (0, 0)),
    ],
    out_shape=[
        jax.ShapeDtypeStruct((1, 1), jnp.float32),
        jax.ShapeDtypeStruct((1, 1), jnp.float32),
    ],
)


def kernel(inputs, weight):
    flat = inputs.reshape(-1, D)
    xsq = jnp.sum(flat**2, axis=1, keepdims=True)
    wsq = jnp.sum(weight**2, axis=1).reshape(1, N_CODES)
    wT = weight.T
    codes, counts, mdist = _dist_argmin(flat, wT, xsq, wsq)
    quant_flat = _sc_gather(weight, codes.reshape(-1))
    loss, perp = _loss_perp(mdist, counts)
    quantized = quant_flat.reshape(inputs.shape)
    return (quantized, loss[0, 0], perp[0, 0], codes)
